# baseline (speedup 1.0000x reference)
import jax
import jax.numpy as jnp
from jax.experimental import pallas as pl
from jax.experimental.pallas import tpu as pltpu

B, N, T, E = 8, 1024, 12, 8
IN_DIM, HID, OUT, K = T * E, 64, 32, 3
G = 2


def _fused_kernel(a_ref, x_ref, w1_ref, b1_ref, w2_ref, b2_ref, w3_ref,
                  b3_ref, out_ref):
    row = jax.lax.broadcasted_iota(jnp.int32, (N, N), 0)
    col = jax.lax.broadcasted_iota(jnp.int32, (N, N), 1)
    diag = row == col

    def matmul(p, q):
        return jax.lax.dot_general(
            p, q, (((1,), (0,)), ((), ())),
            preferred_element_type=jnp.float32)

    a_nds, dinvs = [], []
    for g in range(G):
        a_nd = jnp.where(diag, 0.0, a_ref[g])
        deg = jnp.sum(a_nd, axis=1, keepdims=True)
        dinvs.append(jnp.where(deg > 0,
                               jax.lax.rsqrt(jnp.maximum(deg, 1e-12)), 0.0))
        a_nds.append(a_nd.astype(jnp.bfloat16))

    def cheb(hs, wcat_ref, b_ref, width, last):
        outs = []
        for g in range(G):
            h, a_nd, dinv = hs[g], a_nds[g], dinvs[g]

            def neg_l(p):
                return dinv * matmul(a_nd, (dinv * p).astype(jnp.bfloat16))

            r = matmul(h.astype(jnp.bfloat16), wcat_ref[0])
            r0 = r[:, :width]
            r1 = r[:, width:2 * width]
            r2 = r[:, 2 * width:]
            s = r1 - 2.0 * neg_l(r2)
            o = r0 - r2 - neg_l(s) + b_ref[0]
            outs.append(o if last else jnp.maximum(o, 0.0))
        return outs

    hs = [x_ref[g] for g in range(G)]
    hs = cheb(hs, w1_ref, b1_ref, HID, False)
    hs = cheb(hs, w2_ref, b2_ref, HID, False)
    hs = cheb(hs, w3_ref, b3_ref, OUT, True)
    for g in range(G):
        out_ref[g] = hs[g]


def kernel(X, A, W1, b1, W2, b2, W3, b3):
    x = X.reshape(B, N, IN_DIM)
    b1r = b1.reshape(1, HID)
    b2r = b2.reshape(1, HID)
    b3r = b3.reshape(1, OUT)
    w1c = jnp.concatenate([W1[0], W1[1], W1[2]], axis=1)[None]
    w2c = jnp.concatenate([W2[0], W2[1], W2[2]], axis=1)[None]
    w3c = jnp.concatenate([W3[0], W3[1], W3[2]], axis=1)[None]

    full = lambda *s: pl.BlockSpec(s, lambda b: (0,) * len(s))
    return pl.pallas_call(
        _fused_kernel,
        grid=(B // G,),
        in_specs=[
            pl.BlockSpec((G, N, N), lambda b: (b, 0, 0)),
            pl.BlockSpec((G, N, IN_DIM), lambda b: (b, 0, 0)),
            full(1, IN_DIM, 3 * HID),
            full(1, HID),
            full(1, HID, 3 * HID),
            full(1, HID),
            full(1, HID, 3 * OUT),
            full(1, OUT),
        ],
        out_specs=pl.BlockSpec((G, N, OUT), lambda b: (b, 0, 0)),
        out_shape=jax.ShapeDtypeStruct((B, N, OUT), jnp.float32),
        compiler_params=pltpu.CompilerParams(
            dimension_semantics=("arbitrary",),
        ),
    )(A, x, w1c, b1r, w2c, b2r, w3c, b3r)


# speedup vs baseline: 1.0653x; 1.0653x over previous
import jax
import jax.numpy as jnp
from jax.experimental import pallas as pl
from jax.experimental.pallas import tpu as pltpu

B, N, T, E = 8, 1024, 12, 8
IN_DIM, HID, OUT, K = T * E, 64, 32, 3
G = 2


def _fused_kernel(a_ref, x_ref, w1_ref, b1_ref, w2_ref, b2_ref, w3_ref,
                  b3_ref, out_ref):
    row = jax.lax.broadcasted_iota(jnp.int32, (N, N), 0)
    col = jax.lax.broadcasted_iota(jnp.int32, (N, N), 1)
    diag = row == col

    def matmul(p, q):
        return jax.lax.dot_general(
            p, q, (((1,), (0,)), ((), ())),
            preferred_element_type=jnp.float32)

    Ls = []
    for g in range(G):
        a_nd = jnp.where(diag, 0.0, a_ref[g])
        deg = jnp.sum(a_nd, axis=1, keepdims=True)
        dinv = jnp.where(deg > 0, jax.lax.rsqrt(jnp.maximum(deg, 1e-12)),
                         0.0)
        Ls.append(((-dinv * a_nd) * dinv.reshape(1, N)).astype(jnp.bfloat16))

    def cheb(hs, w_ref, b_ref, last):
        outs = []
        for g in range(G):
            h, L = hs[g], Ls[g]
            t = matmul(h, w_ref[2])
            lt = matmul(L, t.astype(jnp.bfloat16))
            s = matmul(h, w_ref[1]) + 2.0 * lt
            ls = matmul(L, s.astype(jnp.bfloat16))
            o = matmul(h, w_ref[0]) - t + ls + b_ref[0]
            outs.append(o if last else jnp.maximum(o, 0.0))
        return outs

    hs = [x_ref[g] for g in range(G)]
    hs = cheb(hs, w1_ref, b1_ref, False)
    hs = cheb(hs, w2_ref, b2_ref, False)
    hs = cheb(hs, w3_ref, b3_ref, True)
    for g in range(G):
        out_ref[g] = hs[g]


def kernel(X, A, W1, b1, W2, b2, W3, b3):
    x = X.reshape(B, N, IN_DIM)
    b1r = b1.reshape(1, HID)
    b2r = b2.reshape(1, HID)
    b3r = b3.reshape(1, OUT)

    full = lambda *s: pl.BlockSpec(s, lambda b: (0,) * len(s))
    return pl.pallas_call(
        _fused_kernel,
        grid=(B // G,),
        in_specs=[
            pl.BlockSpec((G, N, N), lambda b: (b, 0, 0)),
            pl.BlockSpec((G, N, IN_DIM), lambda b: (b, 0, 0)),
            full(K, IN_DIM, HID),
            full(1, HID),
            full(K, HID, HID),
            full(1, HID),
            full(K, HID, OUT),
            full(1, OUT),
        ],
        out_specs=pl.BlockSpec((G, N, OUT), lambda b: (b, 0, 0)),
        out_shape=jax.ShapeDtypeStruct((B, N, OUT), jnp.float32),
        compiler_params=pltpu.CompilerParams(
            dimension_semantics=("arbitrary",),
        ),
    )(A, x, W1, b1r, W2, b2r, W3, b3r)


# speedup vs baseline: 1.0784x; 1.0123x over previous
import jax
import jax.numpy as jnp
from jax.experimental import pallas as pl
from jax.experimental.pallas import tpu as pltpu

B, N, T, E = 8, 1024, 12, 8
IN_DIM, HID, OUT, K = T * E, 64, 32, 3
G = 2


def _fused_kernel(a_ref, x_ref, w1_ref, b1_ref, w2_ref, b2_ref, w3_ref,
                  b3_ref, out_ref):
    row = jax.lax.broadcasted_iota(jnp.int32, (N, N), 0)
    col = jax.lax.broadcasted_iota(jnp.int32, (N, N), 1)
    diag = row == col

    def matmul(p, q):
        return jax.lax.dot_general(
            p, q, (((1,), (0,)), ((), ())),
            preferred_element_type=jnp.float32)

    Ls = []
    for g in range(G):
        a_nd = jnp.where(diag, 0.0, a_ref[g])
        deg = jnp.sum(a_nd, axis=1, keepdims=True)
        dinv = jnp.where(deg > 0, jax.lax.rsqrt(jnp.maximum(deg, 1e-12)),
                         0.0)
        Ls.append(((-dinv * a_nd) * dinv.reshape(1, N)).astype(jnp.bfloat16))

    def cheb(hs, w_ref, b_ref, last):
        outs = []
        for g in range(G):
            h, L = hs[g], Ls[g]
            hb = h.astype(jnp.bfloat16)
            t = matmul(hb, w_ref[2])
            lt = matmul(L, t.astype(jnp.bfloat16))
            s = matmul(hb, w_ref[1]) + 2.0 * lt
            ls = matmul(L, s.astype(jnp.bfloat16))
            o = matmul(hb, w_ref[0]) - t + ls + b_ref[0]
            outs.append(o if last else jnp.maximum(o, 0.0))
        return outs

    hs = [x_ref[g] for g in range(G)]
    hs = cheb(hs, w1_ref, b1_ref, False)
    hs = cheb(hs, w2_ref, b2_ref, False)
    hs = cheb(hs, w3_ref, b3_ref, True)
    for g in range(G):
        out_ref[g] = hs[g]


def kernel(X, A, W1, b1, W2, b2, W3, b3):
    x = X.reshape(B, N, IN_DIM)
    b1r = b1.reshape(1, HID)
    b2r = b2.reshape(1, HID)
    b3r = b3.reshape(1, OUT)

    full = lambda *s: pl.BlockSpec(s, lambda b: (0,) * len(s))
    return pl.pallas_call(
        _fused_kernel,
        grid=(B // G,),
        in_specs=[
            pl.BlockSpec((G, N, N), lambda b: (b, 0, 0)),
            pl.BlockSpec((G, N, IN_DIM), lambda b: (b, 0, 0)),
            full(K, IN_DIM, HID),
            full(1, HID),
            full(K, HID, HID),
            full(1, HID),
            full(K, HID, OUT),
            full(1, OUT),
        ],
        out_specs=pl.BlockSpec((G, N, OUT), lambda b: (b, 0, 0)),
        out_shape=jax.ShapeDtypeStruct((B, N, OUT), jnp.float32),
        compiler_params=pltpu.CompilerParams(
            dimension_semantics=("arbitrary",),
        ),
    )(A, x, W1, b1r, W2, b2r, W3, b3r)


# speedup vs baseline: 1.3505x; 1.2523x over previous
import jax
import jax.numpy as jnp
from jax.experimental import pallas as pl
from jax.experimental.pallas import tpu as pltpu

B, N, T, E = 8, 1024, 12, 8
IN_DIM, HID, OUT, K = T * E, 64, 32, 3
G = 2


def _fused_kernel(a_ref, x_ref, w1_ref, b1_ref, w2_ref, b2_ref, w3_ref,
                  b3_ref, out_ref):
    row = jax.lax.broadcasted_iota(jnp.int32, (N, N), 0)
    col = jax.lax.broadcasted_iota(jnp.int32, (N, N), 1)
    diag = row == col

    def matmul(p, q):
        return jax.lax.dot_general(
            p, q, (((1,), (0,)), ((), ())),
            preferred_element_type=jnp.float32)

    Ls = []
    for g in range(G):
        a_nd = jnp.where(diag, 0.0, a_ref[g])
        deg = jnp.sum(a_nd, axis=1, keepdims=True)
        dinv = jnp.where(deg > 0, jax.lax.rsqrt(jnp.maximum(deg, 1e-12)),
                         0.0)
        Ls.append(((-dinv * a_nd) * dinv.reshape(1, N)).astype(jnp.bfloat16))

    def cheb(hs, w_ref, b_ref, last):
        w02 = w_ref[0] - w_ref[2]
        outs = []
        for g in range(G):
            hb = hs[g].astype(jnp.bfloat16)
            u = matmul(Ls[g], hb)
            ub = u.astype(jnp.bfloat16)
            v = matmul(ub, w_ref[2])
            o = (matmul(hb, w02) + matmul(ub, w_ref[1])
                 + 2.0 * matmul(Ls[g], v.astype(jnp.bfloat16)) + b_ref[0])
            outs.append(o if last else jnp.maximum(o, 0.0))
        return outs

    hs = [x_ref[g] for g in range(G)]
    hs = cheb(hs, w1_ref, b1_ref, False)
    hs = cheb(hs, w2_ref, b2_ref, False)
    hs = cheb(hs, w3_ref, b3_ref, True)
    for g in range(G):
        out_ref[g] = hs[g]


def kernel(X, A, W1, b1, W2, b2, W3, b3):
    x = X.reshape(B, N, IN_DIM)
    b1r = b1.reshape(1, HID)
    b2r = b2.reshape(1, HID)
    b3r = b3.reshape(1, OUT)

    full = lambda *s: pl.BlockSpec(s, lambda b: (0,) * len(s))
    return pl.pallas_call(
        _fused_kernel,
        grid=(B // G,),
        in_specs=[
            pl.BlockSpec((G, N, N), lambda b: (b, 0, 0)),
            pl.BlockSpec((G, N, IN_DIM), lambda b: (b, 0, 0)),
            full(K, IN_DIM, HID),
            full(1, HID),
            full(K, HID, HID),
            full(1, HID),
            full(K, HID, OUT),
            full(1, OUT),
        ],
        out_specs=pl.BlockSpec((G, N, OUT), lambda b: (b, 0, 0)),
        out_shape=jax.ShapeDtypeStruct((B, N, OUT), jnp.float32),
        compiler_params=pltpu.CompilerParams(
            dimension_semantics=("arbitrary",),
        ),
    )(A, x, W1, b1r, W2, b2r, W3, b3r)
